# Initial kernel scaffold; baseline (speedup 1.0000x reference)
import functools

import jax
import jax.numpy as jnp
from jax import lax
from jax.experimental import pallas as pl
from jax.experimental.pallas import tpu as pltpu
from jax.experimental.pallas import tpu_sc as plsc

NC = 2
NS = 16
NW = NC * NS
CH = 128


def _deg_body(np_, k, col_hbm, ew_hbm, z_hbm, degp_hbm, col_v, ew_v, deg_sh):
    cid = lax.axis_index("c")
    sid = lax.axis_index("s")
    wid = sid * NC + cid
    rps = np_ // NS
    pltpu.sync_copy(z_hbm.at[pl.ds(sid * rps, rps)],
                    deg_sh.at[pl.ds(sid * rps, rps)])
    plsc.subcore_barrier()
    pltpu.sync_copy(col_hbm.at[wid], col_v)
    pltpu.sync_copy(ew_hbm.at[wid], ew_v)

    @pl.loop(0, k)
    def _(j):
        pltpu.sync_copy(ew_v.at[j], deg_sh.at[col_v.at[j]], add=True)

    plsc.subcore_barrier()
    pltpu.sync_copy(deg_sh.at[pl.ds(sid * rps, rps)],
                    degp_hbm.at[cid, pl.ds(sid * rps, rps)])


def _agg_body(np_, k, h, g_hbm, row_hbm, col_hbm, ew_hbm, z_hbm, accp_hbm,
              row_v, col_v, ew_v, buf0, buf1, acc_sh, gsem):
    cid = lax.axis_index("c")
    sid = lax.axis_index("s")
    wid = sid * NC + cid
    rps = np_ // NS
    pltpu.sync_copy(z_hbm.at[pl.ds(sid * rps, rps)],
                    acc_sh.at[pl.ds(sid * rps, rps)])
    plsc.subcore_barrier()
    pltpu.sync_copy(row_hbm.at[wid], row_v)
    pltpu.sync_copy(col_hbm.at[wid], col_v)
    pltpu.sync_copy(ew_hbm.at[wid], ew_v)

    bufs = (buf0, buf1)
    pltpu.async_copy(g_hbm.at[row_v.at[0]], buf0, gsem)

    @pl.loop(0, k, step=2)
    def _(j):
        for b in range(2):
            c = j + b
            pltpu.make_async_copy(g_hbm.at[row_v.at[c]], bufs[b], gsem).wait()

            @pl.when(c + 1 < k)
            def _():
                pltpu.async_copy(g_hbm.at[row_v.at[c + 1]], bufs[1 - b], gsem)

            def scale8(e8, carry):
                for t in range(8):
                    ee = e8 * 8 + t
                    w = ew_v[c, ee, 0]
                    for f in range(h // 16):
                        sl = bufs[b][ee, pl.ds(f * 16, 16)]
                        bufs[b][ee, pl.ds(f * 16, 16)] = sl * w
                return carry

            lax.fori_loop(0, CH // 8, scale8, 0)
            pltpu.sync_copy(bufs[b], acc_sh.at[col_v.at[c]], add=True)

    plsc.subcore_barrier()
    pltpu.sync_copy(acc_sh.at[pl.ds(sid * rps, rps)],
                    accp_hbm.at[cid, pl.ds(sid * rps, rps)])


def _dense1_body(n, x_ref, w1_ref, b1_ref, wc_ref, degp_ref,
                 g_ref, h2_ref, dinv_ref):
    h1 = jnp.maximum(
        jnp.dot(x_ref[...], w1_ref[...], preferred_element_type=jnp.float32)
        + b1_ref[...], 0.0)
    h2 = jnp.dot(h1, wc_ref[...], preferred_element_type=jnp.float32)
    deg = degp_ref[0, :n, :] + degp_ref[1, :n, :] + 1.0
    dinv = jnp.where(deg > 0, lax.rsqrt(deg), 0.0)
    h2_ref[...] = h2
    dinv_ref[...] = dinv
    g_ref[...] = h2 * dinv


def _final_body(n, accp_ref, dinv_ref, h2_ref, bc_ref, w2_ref, b2_ref, out_ref):
    dinv = dinv_ref[...]
    acc = accp_ref[0, :n, :] + accp_ref[1, :n, :]
    s = dinv * acc + (dinv * dinv) * h2_ref[...] + bc_ref[...]
    s = jnp.maximum(s, 0.0)
    o = jnp.dot(s, w2_ref[...], preferred_element_type=jnp.float32) + b2_ref[...]
    m = jnp.max(o, axis=-1, keepdims=True)
    lse = jnp.log(jnp.sum(jnp.exp(o - m), axis=-1, keepdims=True)) + m
    out_ref[...] = o - lse


def kernel(x, edge_index, edge_weight, W1, b1, Wc, bc, W2, b2):
    n, d = x.shape
    h = W1.shape[1]
    ncls = W2.shape[1]
    e = edge_index.shape[1]

    k = -(-e // (NW * CH))
    if k % 2:
        k += 1
    ep = NW * k * CH
    np_ = -(-n // (NS * 8)) * (NS * 8)

    row = edge_index[0]
    col = edge_index[1]
    pad = ep - e
    rowp = jnp.pad(row, (0, pad)).reshape(NW, k, CH)
    colp = jnp.pad(col, (0, pad)).reshape(NW, k, CH)
    ewp = jnp.pad(edge_weight, (0, pad)).reshape(NW, k, CH, 1)
    zcol = jnp.zeros((np_, 1), jnp.float32)
    zfull = jnp.zeros((np_, h), jnp.float32)

    mesh = plsc.VectorSubcoreMesh(core_axis_name="c", subcore_axis_name="s")

    deg_kernel = pl.kernel(
        functools.partial(_deg_body, np_, k),
        out_type=jax.ShapeDtypeStruct((NC, np_, 1), jnp.float32),
        mesh=mesh,
        scratch_types=[
            pltpu.VMEM((k, CH), jnp.int32),
            pltpu.VMEM((k, CH, 1), jnp.float32),
            pltpu.VMEM_SHARED((np_, 1), jnp.float32),
        ],
    )
    degp = deg_kernel(colp, ewp, zcol)

    g, h2, dinv = pl.pallas_call(
        functools.partial(_dense1_body, n),
        out_shape=[
            jax.ShapeDtypeStruct((n, h), jnp.float32),
            jax.ShapeDtypeStruct((n, h), jnp.float32),
            jax.ShapeDtypeStruct((n, 1), jnp.float32),
        ],
    )(x, W1, b1.reshape(1, h), Wc, degp)

    agg_kernel = pl.kernel(
        functools.partial(_agg_body, np_, k, h),
        out_type=jax.ShapeDtypeStruct((NC, np_, h), jnp.float32),
        mesh=mesh,
        scratch_types=[
            pltpu.VMEM((k, CH), jnp.int32),
            pltpu.VMEM((k, CH), jnp.int32),
            pltpu.VMEM((k, CH, 1), jnp.float32),
            pltpu.VMEM((CH, h), jnp.float32),
            pltpu.VMEM((CH, h), jnp.float32),
            pltpu.VMEM_SHARED((np_, h), jnp.float32),
            pltpu.SemaphoreType.DMA,
        ],
    )
    accp = agg_kernel(g, rowp, colp, ewp, zfull)

    out = pl.pallas_call(
        functools.partial(_final_body, n),
        out_shape=jax.ShapeDtypeStruct((n, ncls), jnp.float32),
    )(accp, dinv, h2, bc.reshape(1, bc.shape[0]), W2, b2.reshape(1, ncls))
    return out


# speedup vs baseline: 23.2615x; 23.2615x over previous
import functools

import jax
import jax.numpy as jnp
from jax import lax
from jax.experimental import pallas as pl
from jax.experimental.pallas import tpu as pltpu
from jax.experimental.pallas import tpu_sc as plsc

NC = 2
NS = 16
NW = NC * NS
CH = 128


def _stage_flat(flat_hbm, base, n_real, dst_v, sem, zero16):
    pltpu.async_copy(flat_hbm.at[pl.ds(base, n_real)],
                     dst_v.at[pl.ds(0, n_real)], sem)
    npad = dst_v.shape[0] - n_real
    for t in range(npad // 16):
        dst_v[pl.ds(n_real + t * 16, 16)] = zero16


def _stage_rows(flat_hbm, base, km, tailn, dst_v, sem, zero16):
    for j in range(km):
        pltpu.async_copy(flat_hbm.at[pl.ds(base + j * CH, CH)],
                         dst_v.at[j], sem)
    if tailn:
        pltpu.async_copy(flat_hbm.at[pl.ds(base + km * CH, tailn)],
                         dst_v.at[km, pl.ds(0, tailn)], sem)
        for t in range((CH - tailn) // 16):
            dst_v[km, pl.ds(tailn + t * 16, 16)] = zero16


def _drain(flat_hbm, base, km, tailn, dst_v, sem):
    for j in range(km):
        pltpu.make_async_copy(flat_hbm.at[pl.ds(base + j * CH, CH)],
                              dst_v.at[j], sem).wait()
    if tailn:
        pltpu.make_async_copy(flat_hbm.at[pl.ds(base + km * CH, tailn)],
                              dst_v.at[km, pl.ds(0, tailn)], sem).wait()


def _deg_body(np_, epw, km, tailn, e, eidx_hbm, ew_hbm, degp_hbm,
              col_v, ew_v, zbuf, deg_sh, sem):
    cid = lax.axis_index("c")
    sid = lax.axis_index("s")
    wid = sid * NC + cid
    rps = np_ // NS
    kp = km + (1 if tailn else 0)
    zf = jnp.zeros((16,), jnp.float32)
    zi = jnp.zeros((16,), jnp.int32)
    for i in range(zbuf.shape[0] // 16):
        zbuf[pl.ds(i * 16, 16)] = zf
    pltpu.sync_copy(zbuf.at[pl.ds(0, rps)], deg_sh.at[pl.ds(sid * rps, rps)])
    plsc.subcore_barrier()
    _stage_rows(eidx_hbm, e + wid * epw, km, tailn, col_v, sem, zi)
    _stage_flat(ew_hbm, wid * epw, epw, ew_v, sem, zf)
    _drain(eidx_hbm, e + wid * epw, km, tailn, col_v, sem)
    pltpu.make_async_copy(ew_hbm.at[pl.ds(wid * epw, epw)],
                          ew_v.at[pl.ds(0, epw)], sem).wait()

    @pl.loop(0, kp)
    def _(j):
        pltpu.sync_copy(ew_v.at[pl.ds(j * CH, CH)],
                        deg_sh.at[col_v.at[j]], add=True)

    plsc.subcore_barrier()
    pltpu.sync_copy(deg_sh.at[pl.ds(sid * rps, rps)], zbuf.at[pl.ds(0, rps)])
    pltpu.sync_copy(zbuf.at[pl.ds(0, rps)],
                    degp_hbm.at[pl.ds(cid * np_ + sid * rps, rps)])


def _agg_body(np_, epw, km, tailn, e, h, seg, g_hbm, eidx_hbm, ew_hbm,
              accp_hbm, row_s, col_s, ew_s, rowt_v, colt_v, ewt_v,
              buf0, buf1, acc_sh, sem2, gsem):
    cid = lax.axis_index("c")
    sid = lax.axis_index("s")
    wid = sid * NC + cid
    rps = np_ // NS
    nseg = km // seg
    base_r = wid * epw
    base_c = e + wid * epw
    zf = jnp.zeros((16,), jnp.float32)
    zi = jnp.zeros((16,), jnp.int32)

    def zrow(r, carry):
        for f in range(h // 16):
            buf0[r, pl.ds(f * 16, 16)] = zf
        return carry

    lax.fori_loop(0, CH, zrow, 0)
    nfull, rem = rps // CH, rps % CH
    for t in range(nfull):
        pltpu.sync_copy(buf0, acc_sh.at[pl.ds(sid * rps + t * CH, CH)])
    if rem:
        pltpu.sync_copy(buf0.at[pl.ds(0, rem)],
                        acc_sh.at[pl.ds(sid * rps + nfull * CH, rem)])
    plsc.subcore_barrier()

    if tailn:
        pltpu.sync_copy(eidx_hbm.at[pl.ds(base_r + km * CH, tailn)],
                        rowt_v.at[0, pl.ds(0, tailn)])
        pltpu.sync_copy(eidx_hbm.at[pl.ds(base_c + km * CH, tailn)],
                        colt_v.at[0, pl.ds(0, tailn)])
        pltpu.sync_copy(ew_hbm.at[pl.ds(base_r + km * CH, tailn)],
                        ewt_v.at[pl.ds(0, tailn)])
        for t in range((CH - tailn) // 16):
            rowt_v[0, pl.ds(tailn + t * 16, 16)] = zi
            colt_v[0, pl.ds(tailn + t * 16, 16)] = zi
            ewt_v[pl.ds(tailn + t * 16, 16)] = zf

    def seg_start(s, slot):
        for r in range(seg):
            off = (s * seg + r) * CH
            pltpu.async_copy(eidx_hbm.at[pl.ds(base_r + off, CH)],
                             row_s.at[slot, r], sem2)
            pltpu.async_copy(eidx_hbm.at[pl.ds(base_c + off, CH)],
                             col_s.at[slot, r], sem2)
            pltpu.async_copy(ew_hbm.at[pl.ds(base_r + off, CH)],
                             ew_s.at[slot, r], sem2)

    def seg_wait():
        for r in range(seg):
            pltpu.make_async_copy(eidx_hbm.at[pl.ds(base_r, CH)],
                                  row_s.at[0, 0], sem2).wait()
            pltpu.make_async_copy(eidx_hbm.at[pl.ds(base_c, CH)],
                                  col_s.at[0, 0], sem2).wait()
            pltpu.make_async_copy(ew_hbm.at[pl.ds(base_r, CH)],
                                  ew_s.at[0, 0], sem2).wait()

    bufs = (buf0, buf1)

    def gather_start(slot, cloc, b):
        pltpu.async_copy(g_hbm.at[row_s.at[slot, cloc]], bufs[b], gsem)

    def gather_wait(b):
        pltpu.make_async_copy(g_hbm.at[row_s.at[0, 0]], bufs[b], gsem).wait()

    def scale(ld_w, b):
        def scale16(e16, carry):
            wv = ld_w(e16)
            for t in range(16):
                ee = e16 * 16 + t
                w = wv[t]
                for f in range(h // 16):
                    sl = bufs[b][ee, pl.ds(f * 16, 16)]
                    bufs[b][ee, pl.ds(f * 16, 16)] = sl * w
            return carry

        lax.fori_loop(0, CH // 16, scale16, 0)

    seg_start(0, 0)
    seg_wait()
    if nseg > 1:
        seg_start(1, 1)
    gather_start(0, 0, 0)

    @pl.loop(0, km, step=2)
    def _(j):
        for b in range(2):
            c = j + b
            s = c // seg
            cloc = c % seg
            slot = s % 2
            gather_wait(b)

            @pl.when((cloc == 0) & (s >= 1) & (s + 1 < nseg))
            def _():
                seg_start(s + 1, 1 - slot)

            @pl.when((cloc == seg - 1) & (s + 1 < nseg))
            def _():
                seg_wait()

            nc = c + 1

            @pl.when(nc < km)
            def _():
                gather_start((nc // seg) % 2, nc % seg, 1 - b)

            scale(lambda e16: ew_s[slot, cloc, pl.ds(e16 * 16, 16)], b)
            pltpu.sync_copy(bufs[b], acc_sh.at[col_s.at[slot, cloc]], add=True)

    if tailn:
        pltpu.async_copy(g_hbm.at[rowt_v.at[0]], buf0, gsem)
        gather_wait(0)
        scale(lambda e16: ewt_v[pl.ds(e16 * 16, 16)], 0)
        pltpu.sync_copy(buf0, acc_sh.at[colt_v.at[0]], add=True)

    plsc.subcore_barrier()
    for t in range(nfull):
        pltpu.sync_copy(acc_sh.at[pl.ds(sid * rps + t * CH, CH)], buf0)
        pltpu.sync_copy(buf0, accp_hbm.at[cid, pl.ds(sid * rps + t * CH, CH)])
    if rem:
        pltpu.sync_copy(acc_sh.at[pl.ds(sid * rps + nfull * CH, rem)],
                        buf0.at[pl.ds(0, rem)])
        pltpu.sync_copy(buf0.at[pl.ds(0, rem)],
                        accp_hbm.at[cid, pl.ds(sid * rps + nfull * CH, rem)])


def _dense1_body(n, x_ref, w1_ref, b1_ref, wc_ref, degp_ref,
                 g_ref, h2_ref, dinv_ref):
    h1 = jnp.maximum(
        jnp.dot(x_ref[...], w1_ref[...], preferred_element_type=jnp.float32)
        + b1_ref[...], 0.0)
    h2 = jnp.dot(h1, wc_ref[...], preferred_element_type=jnp.float32)
    degt = lax.dot_general(degp_ref[...], jnp.eye(2, dtype=jnp.float32),
                           dimension_numbers=(((0,), (0,)), ((), ())),
                           preferred_element_type=jnp.float32)
    deg = degt[:n, 0:1] + degt[:n, 1:2] + 1.0
    dinv = jnp.where(deg > 0, lax.rsqrt(deg), 0.0)
    h2_ref[...] = h2
    dinv_ref[...] = dinv
    g_ref[...] = h2 * dinv


def _final_body(n, accp_ref, dinv_ref, h2_ref, bc_ref, w2_ref, b2_ref, out_ref):
    dinv = dinv_ref[...]
    acc = accp_ref[0, :n, :] + accp_ref[1, :n, :]
    s = dinv * acc + (dinv * dinv) * h2_ref[...] + bc_ref[...]
    s = jnp.maximum(s, 0.0)
    o = jnp.dot(s, w2_ref[...], preferred_element_type=jnp.float32) + b2_ref[...]
    m = jnp.max(o, axis=-1, keepdims=True)
    lse = jnp.log(jnp.sum(jnp.exp(o - m), axis=-1, keepdims=True)) + m
    out_ref[...] = o - lse


def kernel(x, edge_index, edge_weight, W1, b1, Wc, bc, W2, b2):
    n, d = x.shape
    h = W1.shape[1]
    ncls = W2.shape[1]
    e = edge_index.shape[1]

    assert e % NW == 0 and e % 8 == 0
    epw = e // NW
    km = epw // CH
    if km % 2:
        km -= 1
    tailn = epw - km * CH
    kp = km + (1 if tailn else 0)
    seg = next(s for s in (8, 6, 4, 2, 1) if km % s == 0)
    np_ = -(-n // (NS * 8)) * (NS * 8)

    eflat = edge_index.reshape(2 * e)

    mesh = plsc.VectorSubcoreMesh(core_axis_name="c", subcore_axis_name="s")

    deg_kernel = pl.kernel(
        functools.partial(_deg_body, np_, epw, km, tailn, e),
        out_type=jax.ShapeDtypeStruct((NC * np_,), jnp.float32),
        mesh=mesh,
        scratch_types=[
            pltpu.VMEM((kp, CH), jnp.int32),
            pltpu.VMEM((kp * CH,), jnp.float32),
            pltpu.VMEM((-(-(np_ // NS) // 16) * 16,), jnp.float32),
            pltpu.VMEM_SHARED((np_,), jnp.float32),
            pltpu.SemaphoreType.DMA,
        ],
    )
    degp = deg_kernel(eflat, edge_weight).reshape(NC, np_)

    g, h2, dinv = pl.pallas_call(
        functools.partial(_dense1_body, n),
        out_shape=[
            jax.ShapeDtypeStruct((n, h), jnp.float32),
            jax.ShapeDtypeStruct((n, h), jnp.float32),
            jax.ShapeDtypeStruct((n, 1), jnp.float32),
        ],
    )(x, W1, b1.reshape(1, h), Wc, degp)

    agg_kernel = pl.kernel(
        functools.partial(_agg_body, np_, epw, km, tailn, e, h, seg),
        out_type=jax.ShapeDtypeStruct((NC, np_, h), jnp.float32),
        mesh=mesh,
        scratch_types=[
            pltpu.VMEM((2, seg, CH), jnp.int32),
            pltpu.VMEM((2, seg, CH), jnp.int32),
            pltpu.VMEM((2, seg, CH), jnp.float32),
            pltpu.VMEM((1, CH), jnp.int32),
            pltpu.VMEM((1, CH), jnp.int32),
            pltpu.VMEM((CH,), jnp.float32),
            pltpu.VMEM((CH, h), jnp.float32),
            pltpu.VMEM((CH, h), jnp.float32),
            pltpu.VMEM_SHARED((np_, h), jnp.float32),
            pltpu.SemaphoreType.DMA,
            pltpu.SemaphoreType.DMA,
        ],
    )
    accp = agg_kernel(g, eflat, edge_weight)

    out = pl.pallas_call(
        functools.partial(_final_body, n),
        out_shape=jax.ShapeDtypeStruct((n, ncls), jnp.float32),
    )(accp, dinv, h2, bc.reshape(1, bc.shape[0]), W2, b2.reshape(1, ncls))
    return out


# speedup vs baseline: 23.2680x; 1.0003x over previous
import functools

import jax
import jax.numpy as jnp
from jax import lax
from jax.experimental import pallas as pl
from jax.experimental.pallas import tpu as pltpu
from jax.experimental.pallas import tpu_sc as plsc

NC = 2
NS = 16
NW = NC * NS
CH = 128


def _stage_flat(flat_hbm, base, n_real, dst_v, sem, zero16):
    pltpu.async_copy(flat_hbm.at[pl.ds(base, n_real)],
                     dst_v.at[pl.ds(0, n_real)], sem)
    npad = dst_v.shape[0] - n_real
    for t in range(npad // 16):
        dst_v[pl.ds(n_real + t * 16, 16)] = zero16


def _stage_rows(flat_hbm, base, km, tailn, dst_v, sem, zero16):
    for j in range(km):
        pltpu.async_copy(flat_hbm.at[pl.ds(base + j * CH, CH)],
                         dst_v.at[j], sem)
    if tailn:
        pltpu.async_copy(flat_hbm.at[pl.ds(base + km * CH, tailn)],
                         dst_v.at[km, pl.ds(0, tailn)], sem)
        for t in range((CH - tailn) // 16):
            dst_v[km, pl.ds(tailn + t * 16, 16)] = zero16


def _drain(flat_hbm, base, km, tailn, dst_v, sem):
    for j in range(km):
        pltpu.make_async_copy(flat_hbm.at[pl.ds(base + j * CH, CH)],
                              dst_v.at[j], sem).wait()
    if tailn:
        pltpu.make_async_copy(flat_hbm.at[pl.ds(base + km * CH, tailn)],
                              dst_v.at[km, pl.ds(0, tailn)], sem).wait()


def _deg_body(np_, epw, km, tailn, e, eidx_hbm, ew_hbm, degp_hbm,
              col_v, ew_v, zbuf, deg_sh, sem):
    cid = lax.axis_index("c")
    sid = lax.axis_index("s")
    wid = sid * NC + cid
    rps = np_ // NS
    kp = km + (1 if tailn else 0)
    zf = jnp.zeros((16,), jnp.float32)
    zi = jnp.zeros((16,), jnp.int32)
    for i in range(zbuf.shape[0] // 16):
        zbuf[pl.ds(i * 16, 16)] = zf
    pltpu.sync_copy(zbuf.at[pl.ds(0, rps)], deg_sh.at[pl.ds(sid * rps, rps)])
    plsc.subcore_barrier()
    _stage_rows(eidx_hbm, e + wid * epw, km, tailn, col_v, sem, zi)
    _stage_flat(ew_hbm, wid * epw, epw, ew_v, sem, zf)
    _drain(eidx_hbm, e + wid * epw, km, tailn, col_v, sem)
    pltpu.make_async_copy(ew_hbm.at[pl.ds(wid * epw, epw)],
                          ew_v.at[pl.ds(0, epw)], sem).wait()

    @pl.loop(0, kp)
    def _(j):
        pltpu.sync_copy(ew_v.at[pl.ds(j * CH, CH)],
                        deg_sh.at[col_v.at[j]], add=True)

    plsc.subcore_barrier()
    pltpu.sync_copy(deg_sh.at[pl.ds(sid * rps, rps)], zbuf.at[pl.ds(0, rps)])
    pltpu.sync_copy(zbuf.at[pl.ds(0, rps)],
                    degp_hbm.at[pl.ds(cid * np_ + sid * rps, rps)])


def _agg_body(np_, epw, km, tailn, e, h, seg, g_hbm, eidx_hbm, ew_hbm,
              accp_hbm, row_s, col_s, ew_s, rowt_v, colt_v, ewt_v,
              buf0, buf1, acc_sh, sem2, gsem, ssem):
    cid = lax.axis_index("c")
    sid = lax.axis_index("s")
    wid = sid * NC + cid
    rps = np_ // NS
    nseg = km // seg
    base_r = wid * epw
    base_c = e + wid * epw
    zf = jnp.zeros((16,), jnp.float32)
    zi = jnp.zeros((16,), jnp.int32)

    def zrow(r, carry):
        for f in range(h // 16):
            buf0[r, pl.ds(f * 16, 16)] = zf
        return carry

    lax.fori_loop(0, CH, zrow, 0)
    nfull, rem = rps // CH, rps % CH
    for t in range(nfull):
        pltpu.sync_copy(buf0, acc_sh.at[pl.ds(sid * rps + t * CH, CH)])
    if rem:
        pltpu.sync_copy(buf0.at[pl.ds(0, rem)],
                        acc_sh.at[pl.ds(sid * rps + nfull * CH, rem)])
    plsc.subcore_barrier()

    if tailn:
        pltpu.sync_copy(eidx_hbm.at[pl.ds(base_r + km * CH, tailn)],
                        rowt_v.at[0, pl.ds(0, tailn)])
        pltpu.sync_copy(eidx_hbm.at[pl.ds(base_c + km * CH, tailn)],
                        colt_v.at[0, pl.ds(0, tailn)])
        pltpu.sync_copy(ew_hbm.at[pl.ds(base_r + km * CH, tailn)],
                        ewt_v.at[pl.ds(0, tailn)])
        for t in range((CH - tailn) // 16):
            rowt_v[0, pl.ds(tailn + t * 16, 16)] = zi
            colt_v[0, pl.ds(tailn + t * 16, 16)] = zi
            ewt_v[pl.ds(tailn + t * 16, 16)] = zf

    def seg_start(s, slot):
        for r in range(seg):
            off = (s * seg + r) * CH
            pltpu.async_copy(eidx_hbm.at[pl.ds(base_r + off, CH)],
                             row_s.at[slot, r], sem2)
            pltpu.async_copy(eidx_hbm.at[pl.ds(base_c + off, CH)],
                             col_s.at[slot, r], sem2)
            pltpu.async_copy(ew_hbm.at[pl.ds(base_r + off, CH)],
                             ew_s.at[slot, r], sem2)

    def seg_wait():
        for r in range(seg):
            pltpu.make_async_copy(eidx_hbm.at[pl.ds(base_r, CH)],
                                  row_s.at[0, 0], sem2).wait()
            pltpu.make_async_copy(eidx_hbm.at[pl.ds(base_c, CH)],
                                  col_s.at[0, 0], sem2).wait()
            pltpu.make_async_copy(ew_hbm.at[pl.ds(base_r, CH)],
                                  ew_s.at[0, 0], sem2).wait()

    bufs = (buf0, buf1)

    def gather_start(slot, cloc, b):
        pltpu.async_copy(g_hbm.at[row_s.at[slot, cloc]], bufs[b], gsem)

    def gather_wait(b):
        pltpu.make_async_copy(g_hbm.at[row_s.at[0, 0]], bufs[b], gsem).wait()

    def scale(ld_w, b):
        def scale16(e16, carry):
            wv = ld_w(e16)
            for t in range(16):
                ee = e16 * 16 + t
                w = wv[t]
                for f in range(h // 16):
                    sl = bufs[b][ee, pl.ds(f * 16, 16)]
                    bufs[b][ee, pl.ds(f * 16, 16)] = sl * w
            return carry

        lax.fori_loop(0, CH // 16, scale16, 0)

    seg_start(0, 0)
    seg_wait()
    if nseg > 1:
        seg_start(1, 1)
    gather_start(0, 0, 0)

    def scatter_drain():
        pltpu.make_async_copy(buf0, acc_sh.at[col_s.at[0, 0]], ssem).wait()

    @pl.loop(0, km, step=2)
    def _(j):
        for b in range(2):
            c = j + b
            s = c // seg
            cloc = c % seg
            slot = s % 2
            gather_wait(b)

            @pl.when(c >= 1)
            def _():
                scatter_drain()

            @pl.when((cloc == 0) & (s >= 1) & (s + 1 < nseg))
            def _():
                seg_start(s + 1, 1 - slot)

            @pl.when((cloc == seg - 1) & (s + 1 < nseg))
            def _():
                seg_wait()

            nc = c + 1

            @pl.when(nc < km)
            def _():
                gather_start((nc // seg) % 2, nc % seg, 1 - b)

            scale(lambda e16: ew_s[slot, cloc, pl.ds(e16 * 16, 16)], b)
            pltpu.async_copy(bufs[b], acc_sh.at[col_s.at[slot, cloc]], ssem,
                             add=True)

    if km > 0:
        scatter_drain()
    if tailn:
        pltpu.async_copy(g_hbm.at[rowt_v.at[0]], buf0, gsem)
        gather_wait(0)
        scale(lambda e16: ewt_v[pl.ds(e16 * 16, 16)], 0)
        pltpu.sync_copy(buf0, acc_sh.at[colt_v.at[0]], add=True)

    plsc.subcore_barrier()
    for t in range(nfull):
        pltpu.sync_copy(acc_sh.at[pl.ds(sid * rps + t * CH, CH)], buf0)
        pltpu.sync_copy(buf0, accp_hbm.at[cid, pl.ds(sid * rps + t * CH, CH)])
    if rem:
        pltpu.sync_copy(acc_sh.at[pl.ds(sid * rps + nfull * CH, rem)],
                        buf0.at[pl.ds(0, rem)])
        pltpu.sync_copy(buf0.at[pl.ds(0, rem)],
                        accp_hbm.at[cid, pl.ds(sid * rps + nfull * CH, rem)])


def _dense1_body(n, x_ref, w1_ref, b1_ref, wc_ref, degp_ref,
                 g_ref, h2_ref, dinv_ref):
    h1 = jnp.maximum(
        jnp.dot(x_ref[...], w1_ref[...], preferred_element_type=jnp.float32)
        + b1_ref[...], 0.0)
    h2 = jnp.dot(h1, wc_ref[...], preferred_element_type=jnp.float32)
    degt = lax.dot_general(degp_ref[...], jnp.eye(2, dtype=jnp.float32),
                           dimension_numbers=(((0,), (0,)), ((), ())),
                           preferred_element_type=jnp.float32)
    deg = degt[:n, 0:1] + degt[:n, 1:2] + 1.0
    dinv = jnp.where(deg > 0, lax.rsqrt(deg), 0.0)
    h2_ref[...] = h2
    dinv_ref[...] = dinv
    g_ref[...] = h2 * dinv


def _final_body(n, accp_ref, dinv_ref, h2_ref, bc_ref, w2_ref, b2_ref, out_ref):
    dinv = dinv_ref[...]
    acc = accp_ref[0, :n, :] + accp_ref[1, :n, :]
    s = dinv * acc + (dinv * dinv) * h2_ref[...] + bc_ref[...]
    s = jnp.maximum(s, 0.0)
    o = jnp.dot(s, w2_ref[...], preferred_element_type=jnp.float32) + b2_ref[...]
    m = jnp.max(o, axis=-1, keepdims=True)
    lse = jnp.log(jnp.sum(jnp.exp(o - m), axis=-1, keepdims=True)) + m
    out_ref[...] = o - lse


def kernel(x, edge_index, edge_weight, W1, b1, Wc, bc, W2, b2):
    n, d = x.shape
    h = W1.shape[1]
    ncls = W2.shape[1]
    e = edge_index.shape[1]

    assert e % NW == 0 and e % 8 == 0
    epw = e // NW
    km = epw // CH
    if km % 2:
        km -= 1
    tailn = epw - km * CH
    kp = km + (1 if tailn else 0)
    seg = next(s for s in (8, 6, 4, 2, 1) if km % s == 0)
    np_ = -(-n // (NS * 8)) * (NS * 8)

    eflat = edge_index.reshape(2 * e)

    mesh = plsc.VectorSubcoreMesh(core_axis_name="c", subcore_axis_name="s")

    deg_kernel = pl.kernel(
        functools.partial(_deg_body, np_, epw, km, tailn, e),
        out_type=jax.ShapeDtypeStruct((NC * np_,), jnp.float32),
        mesh=mesh,
        scratch_types=[
            pltpu.VMEM((kp, CH), jnp.int32),
            pltpu.VMEM((kp * CH,), jnp.float32),
            pltpu.VMEM((-(-(np_ // NS) // 16) * 16,), jnp.float32),
            pltpu.VMEM_SHARED((np_,), jnp.float32),
            pltpu.SemaphoreType.DMA,
        ],
    )
    degp = deg_kernel(eflat, edge_weight).reshape(NC, np_)

    g, h2, dinv = pl.pallas_call(
        functools.partial(_dense1_body, n),
        out_shape=[
            jax.ShapeDtypeStruct((n, h), jnp.float32),
            jax.ShapeDtypeStruct((n, h), jnp.float32),
            jax.ShapeDtypeStruct((n, 1), jnp.float32),
        ],
    )(x, W1, b1.reshape(1, h), Wc, degp)

    agg_kernel = pl.kernel(
        functools.partial(_agg_body, np_, epw, km, tailn, e, h, seg),
        out_type=jax.ShapeDtypeStruct((NC, np_, h), jnp.float32),
        mesh=mesh,
        scratch_types=[
            pltpu.VMEM((2, seg, CH), jnp.int32),
            pltpu.VMEM((2, seg, CH), jnp.int32),
            pltpu.VMEM((2, seg, CH), jnp.float32),
            pltpu.VMEM((1, CH), jnp.int32),
            pltpu.VMEM((1, CH), jnp.int32),
            pltpu.VMEM((CH,), jnp.float32),
            pltpu.VMEM((CH, h), jnp.float32),
            pltpu.VMEM((CH, h), jnp.float32),
            pltpu.VMEM_SHARED((np_, h), jnp.float32),
            pltpu.SemaphoreType.DMA,
            pltpu.SemaphoreType.DMA,
            pltpu.SemaphoreType.DMA,
        ],
    )
    accp = agg_kernel(g, eflat, edge_weight)

    out = pl.pallas_call(
        functools.partial(_final_body, n),
        out_shape=jax.ShapeDtypeStruct((n, ncls), jnp.float32),
    )(accp, dinv, h2, bc.reshape(1, bc.shape[0]), W2, b2.reshape(1, ncls))
    return out


# speedup vs baseline: 23.3160x; 1.0021x over previous
import functools

import jax
import jax.numpy as jnp
from jax import lax
from jax.experimental import pallas as pl
from jax.experimental.pallas import tpu as pltpu
from jax.experimental.pallas import tpu_sc as plsc

NC = 2
NS = 16
NW = NC * NS
CH = 128


def _stage_flat(flat_hbm, base, n_real, dst_v, sem, zero16):
    pltpu.async_copy(flat_hbm.at[pl.ds(base, n_real)],
                     dst_v.at[pl.ds(0, n_real)], sem)
    npad = dst_v.shape[0] - n_real
    for t in range(npad // 16):
        dst_v[pl.ds(n_real + t * 16, 16)] = zero16


def _stage_rows(flat_hbm, base, km, tailn, dst_v, sem, zero16):
    for j in range(km):
        pltpu.async_copy(flat_hbm.at[pl.ds(base + j * CH, CH)],
                         dst_v.at[j], sem)
    if tailn:
        pltpu.async_copy(flat_hbm.at[pl.ds(base + km * CH, tailn)],
                         dst_v.at[km, pl.ds(0, tailn)], sem)
        for t in range((CH - tailn) // 16):
            dst_v[km, pl.ds(tailn + t * 16, 16)] = zero16


def _drain(flat_hbm, base, km, tailn, dst_v, sem):
    for j in range(km):
        pltpu.make_async_copy(flat_hbm.at[pl.ds(base + j * CH, CH)],
                              dst_v.at[j], sem).wait()
    if tailn:
        pltpu.make_async_copy(flat_hbm.at[pl.ds(base + km * CH, tailn)],
                              dst_v.at[km, pl.ds(0, tailn)], sem).wait()


def _deg_body(np_, epw, km, tailn, e, eidx_hbm, ew_hbm, degp_hbm,
              col_v, ew_v, zbuf, deg_sh, sem):
    cid = lax.axis_index("c")
    sid = lax.axis_index("s")
    wid = sid * NC + cid
    rps = np_ // NS
    kp = km + (1 if tailn else 0)
    zf = jnp.zeros((16,), jnp.float32)
    zi = jnp.zeros((16,), jnp.int32)
    for i in range(zbuf.shape[0] // 16):
        zbuf[pl.ds(i * 16, 16)] = zf
    pltpu.sync_copy(zbuf.at[pl.ds(0, rps)], deg_sh.at[pl.ds(sid * rps, rps)])
    plsc.subcore_barrier()
    _stage_rows(eidx_hbm, e + wid * epw, km, tailn, col_v, sem, zi)
    _stage_flat(ew_hbm, wid * epw, epw, ew_v, sem, zf)
    _drain(eidx_hbm, e + wid * epw, km, tailn, col_v, sem)
    pltpu.make_async_copy(ew_hbm.at[pl.ds(wid * epw, epw)],
                          ew_v.at[pl.ds(0, epw)], sem).wait()

    @pl.loop(0, kp)
    def _(j):
        pltpu.sync_copy(ew_v.at[pl.ds(j * CH, CH)],
                        deg_sh.at[col_v.at[j]], add=True)

    plsc.subcore_barrier()
    pltpu.sync_copy(deg_sh.at[pl.ds(sid * rps, rps)], zbuf.at[pl.ds(0, rps)])
    pltpu.sync_copy(zbuf.at[pl.ds(0, rps)],
                    degp_hbm.at[pl.ds(cid * np_ + sid * rps, rps)])


def _agg_body(np_, epw, km, tailn, e, h, seg, g_hbm, eidx_hbm, ew_hbm,
              accp_hbm, row_s, col_s, ew_s, rowt_v, colt_v, ewt_v,
              buf0, buf1, acc_sh, sem2, gsem, ssem):
    cid = lax.axis_index("c")
    sid = lax.axis_index("s")
    wid = sid * NC + cid
    rps = np_ // NS
    nseg = km // seg
    base_r = wid * epw
    base_c = e + wid * epw
    zf = jnp.zeros((16,), jnp.float32)
    zi = jnp.zeros((16,), jnp.int32)

    def zrow(r, carry):
        for f in range(h // 16):
            buf0[r, pl.ds(f * 16, 16)] = zf
        return carry

    lax.fori_loop(0, CH, zrow, 0)
    nfull, rem = rps // CH, rps % CH
    for t in range(nfull):
        pltpu.sync_copy(buf0, acc_sh.at[pl.ds(sid * rps + t * CH, CH)])
    if rem:
        pltpu.sync_copy(buf0.at[pl.ds(0, rem)],
                        acc_sh.at[pl.ds(sid * rps + nfull * CH, rem)])
    plsc.subcore_barrier()

    if tailn:
        pltpu.sync_copy(eidx_hbm.at[pl.ds(base_r + km * CH, tailn)],
                        rowt_v.at[0, pl.ds(0, tailn)])
        pltpu.sync_copy(eidx_hbm.at[pl.ds(base_c + km * CH, tailn)],
                        colt_v.at[0, pl.ds(0, tailn)])
        pltpu.sync_copy(ew_hbm.at[pl.ds(base_r + km * CH, tailn)],
                        ewt_v.at[pl.ds(0, tailn)])
        for t in range((CH - tailn) // 16):
            rowt_v[0, pl.ds(tailn + t * 16, 16)] = zi
            colt_v[0, pl.ds(tailn + t * 16, 16)] = zi
            ewt_v[pl.ds(tailn + t * 16, 16)] = zf

    def seg_start(s, slot):
        for r in range(seg):
            off = (s * seg + r) * CH
            pltpu.async_copy(eidx_hbm.at[pl.ds(base_r + off, CH)],
                             row_s.at[slot, r], sem2)
            pltpu.async_copy(eidx_hbm.at[pl.ds(base_c + off, CH)],
                             col_s.at[slot, r], sem2)
            pltpu.async_copy(ew_hbm.at[pl.ds(base_r + off, CH)],
                             ew_s.at[slot, r], sem2)

    def seg_wait():
        for r in range(seg):
            pltpu.make_async_copy(eidx_hbm.at[pl.ds(base_r, CH)],
                                  row_s.at[0, 0], sem2).wait()
            pltpu.make_async_copy(eidx_hbm.at[pl.ds(base_c, CH)],
                                  col_s.at[0, 0], sem2).wait()
            pltpu.make_async_copy(ew_hbm.at[pl.ds(base_r, CH)],
                                  ew_s.at[0, 0], sem2).wait()

    bufs = (buf0, buf1)

    def gather_start(slot, cloc, b):
        pltpu.async_copy(g_hbm.at[row_s.at[slot, cloc]], bufs[b], gsem)

    def gather_wait(b):
        pltpu.make_async_copy(g_hbm.at[row_s.at[0, 0]], bufs[b], gsem).wait()

    def scale(ld_w, b):
        def scale16(e16, carry):
            wv = ld_w(e16)
            for t in range(16):
                ee = e16 * 16 + t
                w = wv[t]
                for f in range(h // 16):
                    sl = bufs[b][ee, pl.ds(f * 16, 16)]
                    bufs[b][ee, pl.ds(f * 16, 16)] = sl * w
            return carry

        lax.fori_loop(0, CH // 16, scale16, 0)

    seg_start(0, 0)
    seg_wait()
    if nseg > 1:
        seg_start(1, 1)
    gather_start(0, 0, 0)

    def scatter_drain():
        pltpu.make_async_copy(buf0, acc_sh.at[col_s.at[0, 0]], ssem).wait()

    @pl.loop(0, km, step=2)
    def _(j):
        for b in range(2):
            c = j + b
            s = c // seg
            cloc = c % seg
            slot = s % 2
            gather_wait(b)

            @pl.when(c >= 1)
            def _():
                scatter_drain()

            @pl.when((cloc == 0) & (s >= 1) & (s + 1 < nseg))
            def _():
                seg_start(s + 1, 1 - slot)

            @pl.when((cloc == seg - 1) & (s + 1 < nseg))
            def _():
                seg_wait()

            nc = c + 1

            @pl.when(nc < km)
            def _():
                gather_start((nc // seg) % 2, nc % seg, 1 - b)

            scale(lambda e16: ew_s[slot, cloc, pl.ds(e16 * 16, 16)], b)
            pltpu.async_copy(bufs[b], acc_sh.at[col_s.at[slot, cloc]], ssem,
                             add=True)

    if km > 0:
        scatter_drain()
    if tailn:
        pltpu.async_copy(g_hbm.at[rowt_v.at[0]], buf0, gsem)
        gather_wait(0)
        scale(lambda e16: ewt_v[pl.ds(e16 * 16, 16)], 0)
        pltpu.sync_copy(buf0, acc_sh.at[colt_v.at[0]], add=True)

    plsc.subcore_barrier()
    for t in range(nfull):
        pltpu.sync_copy(acc_sh.at[pl.ds(sid * rps + t * CH, CH)], buf0)
        pltpu.sync_copy(buf0, accp_hbm.at[cid, pl.ds(sid * rps + t * CH, CH)])
    if rem:
        pltpu.sync_copy(acc_sh.at[pl.ds(sid * rps + nfull * CH, rem)],
                        buf0.at[pl.ds(0, rem)])
        pltpu.sync_copy(buf0.at[pl.ds(0, rem)],
                        accp_hbm.at[cid, pl.ds(sid * rps + nfull * CH, rem)])


def _h2_body(x_ref, w1_ref, b1_ref, wc_ref, h2_ref):
    h1 = jnp.maximum(
        jnp.dot(x_ref[...], w1_ref[...], preferred_element_type=jnp.float32)
        + b1_ref[...], 0.0)
    h2_ref[...] = jnp.dot(h1, wc_ref[...], preferred_element_type=jnp.float32)


def _g_body(n, h2_ref, degp_ref, g_ref, dinv_ref):
    degt = lax.dot_general(degp_ref[...], jnp.eye(2, dtype=jnp.float32),
                           dimension_numbers=(((0,), (0,)), ((), ())),
                           preferred_element_type=jnp.float32)
    deg = degt[:n, 0:1] + degt[:n, 1:2] + 1.0
    dinv = jnp.where(deg > 0, lax.rsqrt(deg), 0.0)
    dinv_ref[...] = dinv
    g_ref[...] = h2_ref[...] * dinv


def _final_body(n, accp_ref, dinv_ref, h2_ref, bc_ref, w2_ref, b2_ref, out_ref):
    dinv = dinv_ref[...]
    acc = accp_ref[0, :n, :] + accp_ref[1, :n, :]
    s = dinv * acc + (dinv * dinv) * h2_ref[...] + bc_ref[...]
    s = jnp.maximum(s, 0.0)
    o = jnp.dot(s, w2_ref[...], preferred_element_type=jnp.float32) + b2_ref[...]
    m = jnp.max(o, axis=-1, keepdims=True)
    lse = jnp.log(jnp.sum(jnp.exp(o - m), axis=-1, keepdims=True)) + m
    out_ref[...] = o - lse


def kernel(x, edge_index, edge_weight, W1, b1, Wc, bc, W2, b2):
    n, d = x.shape
    h = W1.shape[1]
    ncls = W2.shape[1]
    e = edge_index.shape[1]

    assert e % NW == 0 and e % 8 == 0
    epw = e // NW
    km = epw // CH
    if km % 2:
        km -= 1
    tailn = epw - km * CH
    kp = km + (1 if tailn else 0)
    seg = next(s for s in (8, 6, 4, 2, 1) if km % s == 0)
    np_ = -(-n // (NS * 8)) * (NS * 8)

    eflat = edge_index.reshape(2 * e)

    mesh = plsc.VectorSubcoreMesh(core_axis_name="c", subcore_axis_name="s")

    deg_kernel = pl.kernel(
        functools.partial(_deg_body, np_, epw, km, tailn, e),
        out_type=jax.ShapeDtypeStruct((NC * np_,), jnp.float32),
        mesh=mesh,
        scratch_types=[
            pltpu.VMEM((kp, CH), jnp.int32),
            pltpu.VMEM((kp * CH,), jnp.float32),
            pltpu.VMEM((-(-(np_ // NS) // 16) * 16,), jnp.float32),
            pltpu.VMEM_SHARED((np_,), jnp.float32),
            pltpu.SemaphoreType.DMA,
        ],
    )
    degp = deg_kernel(eflat, edge_weight).reshape(NC, np_)

    h2 = pl.pallas_call(
        _h2_body,
        out_shape=jax.ShapeDtypeStruct((n, h), jnp.float32),
    )(x, W1, b1.reshape(1, h), Wc)

    g, dinv = pl.pallas_call(
        functools.partial(_g_body, n),
        out_shape=[
            jax.ShapeDtypeStruct((n, h), jnp.float32),
            jax.ShapeDtypeStruct((n, 1), jnp.float32),
        ],
    )(h2, degp)

    agg_kernel = pl.kernel(
        functools.partial(_agg_body, np_, epw, km, tailn, e, h, seg),
        out_type=jax.ShapeDtypeStruct((NC, np_, h), jnp.float32),
        mesh=mesh,
        scratch_types=[
            pltpu.VMEM((2, seg, CH), jnp.int32),
            pltpu.VMEM((2, seg, CH), jnp.int32),
            pltpu.VMEM((2, seg, CH), jnp.float32),
            pltpu.VMEM((1, CH), jnp.int32),
            pltpu.VMEM((1, CH), jnp.int32),
            pltpu.VMEM((CH,), jnp.float32),
            pltpu.VMEM((CH, h), jnp.float32),
            pltpu.VMEM((CH, h), jnp.float32),
            pltpu.VMEM_SHARED((np_, h), jnp.float32),
            pltpu.SemaphoreType.DMA,
            pltpu.SemaphoreType.DMA,
            pltpu.SemaphoreType.DMA,
        ],
    )
    accp = agg_kernel(g, eflat, edge_weight)

    out = pl.pallas_call(
        functools.partial(_final_body, n),
        out_shape=jax.ShapeDtypeStruct((n, ncls), jnp.float32),
    )(accp, dinv, h2, bc.reshape(1, bc.shape[0]), W2, b2.reshape(1, ncls))
    return out
